# baseline (speedup 1.0000x reference)
import dataclasses
import functools

import jax
import jax.numpy as jnp
from jax import lax
from jax.experimental import pallas as pl
from jax.experimental.pallas import tpu as pltpu
from jax.experimental.pallas import tpu_sc as plsc

N = 50000
E = 800000
H = 64

HALF = 25088
NPAD = 2 * HALF
ACC_ROWS = 25216
ZW = NPAD // 128
ZCH = 49

EPAD = 819200
EW = EPAD // 128
WPT_B = EW // 16
CH_B = 8
NCH_B = WPT_B // CH_B
SW = 2 * CH_B
WPT_A = EW // 32
CH_A = 8
NCH_A = WPT_A // CH_A

_MESH = plsc.VectorSubcoreMesh(
    core_axis_name="c", subcore_axis_name="s", num_cores=2, num_subcores=16)

_CP = pltpu.CompilerParams(
    needs_layout_passes=False, use_tc_tiling_on_sc=False)


def _embed_body(z_hbm, emb_hbm, x_hbm, zv, gb):
    wid = lax.axis_index("s") * 2 + lax.axis_index("c")
    for i in range(2):
        ci = wid + 32 * i

        def _one(ci=ci):
            pltpu.sync_copy(z_hbm.at[pl.ds(ci * 8, 8)], zv)
            for w in range(8):
                pltpu.sync_copy(emb_hbm.at[zv.at[w]], gb)
                pltpu.sync_copy(gb, x_hbm.at[pl.ds((ci * 8 + w) * 128, 128)])

        if i == 0:
            _one()
        else:
            pl.when(ci < ZCH)(_one)


def _embed(z2, emb):
    f = pl.kernel(
        _embed_body,
        out_type=jax.ShapeDtypeStruct((NPAD, H), jnp.float32),
        mesh=_MESH,
        compiler_params=_CP,
        scratch_types=[
            pltpu.VMEM((8, 128), jnp.int32),
            pltpu.VMEM((128, H), jnp.float32),
        ],
    )
    return f(z2, emb)


def _rsqrt(s):
    i = plsc.bitcast(s, jnp.int32)
    y = plsc.bitcast(jnp.int32(0x5F3759DF) - (i >> 1), jnp.float32)
    for _ in range(3):
        y = y * (1.5 - 0.5 * s * y * y)
    return y


def _rbf_body(row_hbm, col_hbm, px_hbm, py_hbm, pz_hbm, rbf_hbm,
              rv, cv, bxr, byr, bzr, bxc, byc, bzc, ob, sem):
    wid = lax.axis_index("s") * 2 + lax.axis_index("c")
    base_w = wid * WPT_A

    @pl.loop(0, NCH_A)
    def _chunk(ch):
        w0 = base_w + ch * CH_A
        pltpu.sync_copy(row_hbm.at[pl.ds(w0, CH_A)], rv)
        pltpu.sync_copy(col_hbm.at[pl.ds(w0, CH_A)], cv)
        cps = []
        for w in range(CH_A):
            cps.append(pltpu.async_copy(px_hbm.at[rv.at[w]], bxr.at[w], sem))
            cps.append(pltpu.async_copy(py_hbm.at[rv.at[w]], byr.at[w], sem))
            cps.append(pltpu.async_copy(pz_hbm.at[rv.at[w]], bzr.at[w], sem))
            cps.append(pltpu.async_copy(px_hbm.at[cv.at[w]], bxc.at[w], sem))
            cps.append(pltpu.async_copy(py_hbm.at[cv.at[w]], byc.at[w], sem))
            cps.append(pltpu.async_copy(pz_hbm.at[cv.at[w]], bzc.at[w], sem))
        for cp in cps:
            cp.wait()
        for w in range(CH_A):
            for j in range(0, 128, 16):
                sl = (w, pl.ds(j, 16))
                dx = bxr[sl] - bxc[sl]
                dy = byr[sl] - byc[sl]
                dz = bzr[sl] - bzc[sl]
                s = dx * dx + dy * dy + dz * dz + 1e-12
                r = jnp.exp(-(s * _rsqrt(s)))
                eidx = (w0 + w) * 128 + j + lax.iota(jnp.int32, 16)
                ob[sl] = jnp.where(eidx < E, r, 0.0)
        pltpu.sync_copy(ob, rbf_hbm.at[pl.ds(w0, CH_A)])


def _rbf(row2, col2, px, py, pz):
    f = pl.kernel(
        _rbf_body,
        out_type=jax.ShapeDtypeStruct((EW, 128), jnp.float32),
        mesh=_MESH,
        compiler_params=_CP,
        scratch_types=(
            [pltpu.VMEM((CH_A, 128), jnp.int32)] * 2
            + [pltpu.VMEM((CH_A, 128), jnp.float32)] * 7
            + [pltpu.SemaphoreType.DMA]
        ),
    )
    return f(row2, col2, px, py, pz)


def _mp_body(x_hbm, row_hbm, col_hbm, rbf_hbm, zero_hbm, out_hbm,
             rv, cv, fv, iv2, gb, acc,
             g0, g1, g2, g3, t0, t1, t2, t3):
    gsem = (g0, g1, g2, g3)
    tsem = (t0, t1, t2, t3)
    c = lax.axis_index("c")
    s = lax.axis_index("s")
    nbase = c * HALF
    zrows = ACC_ROWS // 16
    pltpu.sync_copy(zero_hbm.at[pl.ds(0, zrows)],
                    acc.at[pl.ds(s * zrows, zrows)])
    plsc.subcore_barrier()
    w_base = s * WPT_B

    @pl.loop(0, NCH_B)
    def _chunk(ch):
        w0 = w_base + ch * CH_B
        pltpu.sync_copy(row_hbm.at[pl.ds(w0, CH_B)], rv)
        pltpu.sync_copy(col_hbm.at[pl.ds(w0, CH_B)], cv)
        gps = [None] * 4
        for q in range(3):
            gps[q] = pltpu.async_copy(
                x_hbm.at[cv.at[q // 2, pl.ds((q % 2) * 64, 64)]],
                gb.at[q], gsem[q])
        pltpu.sync_copy(rbf_hbm.at[pl.ds(w0, CH_B)], fv)
        for q in range(SW):
            w8, h = q // 2, (q % 2) * 64
            for j in range(0, 64, 16):
                r16 = rv[w8, pl.ds(h + j, 16)]
                loc = r16 - nbase
                ok = (loc >= 0) & (loc < HALF)
                iv2[q, pl.ds(j, 16)] = jnp.where(ok, loc, HALF)
        scs = [None] * 4
        for q in range(SW):
            b = q % 4
            nq = q + 3
            if nq < SW:
                nb = nq % 4
                if scs[nb] is not None:
                    scs[nb].wait()
                gps[nb] = pltpu.async_copy(
                    x_hbm.at[cv.at[nq // 2, pl.ds((nq % 2) * 64, 64)]],
                    gb.at[nb], gsem[nb])
            gps[b].wait()

            @pl.loop(0, 64, step=16)
            def _scale(j, b=b, q=q):
                f16 = fv[q // 2, pl.ds((q % 2) * 64 + j, 16)]
                for u in range(16):
                    rb = f16[u]
                    for k in range(4):
                        sl = (b, j + u, pl.ds(k * 16, 16))
                        gb[sl] = gb[sl] * rb

            scs[b] = pltpu.async_copy(gb.at[b], acc.at[iv2.at[q]],
                                      tsem[b], add=True)
        for b in range(4):
            scs[b].wait()

    plsc.subcore_barrier()
    rpt = HALF // 16
    pltpu.sync_copy(acc.at[pl.ds(s * rpt, rpt)],
                    out_hbm.at[pl.ds(nbase + s * rpt, rpt)])


def _mp(x, row2, col2, rbf2, zeros):
    f = pl.kernel(
        _mp_body,
        out_type=jax.ShapeDtypeStruct((NPAD, H), jnp.float32),
        mesh=_MESH,
        compiler_params=_CP,
        scratch_types=[
            pltpu.VMEM((CH_B, 128), jnp.int32),
            pltpu.VMEM((CH_B, 128), jnp.int32),
            pltpu.VMEM((CH_B, 128), jnp.float32),
            pltpu.VMEM((SW, 64), jnp.int32),
            pltpu.VMEM((4, 64, H), jnp.float32),
            pltpu.VMEM_SHARED((ACC_ROWS, H), jnp.float32),
        ] + [pltpu.SemaphoreType.DMA] * 8,
    )
    return f(x, row2, col2, rbf2, zeros)


def _mm_body(x_ref, o_ref, w_ref, b_ref, out_ref):
    acc = x_ref[...] + o_ref[...]
    out_ref[...] = jnp.maximum(
        lax.dot_general(acc, w_ref[...], (((1,), (1,)), ((), ())),
                        preferred_element_type=jnp.float32) + b_ref[...], 0.0)


def _mm(x, out, W, b):
    blk = 1024
    return pl.pallas_call(
        _mm_body,
        grid=(NPAD // blk,),
        in_specs=[
            pl.BlockSpec((blk, H), lambda i: (i, 0)),
            pl.BlockSpec((blk, H), lambda i: (i, 0)),
            pl.BlockSpec((H, H), lambda i: (0, 0)),
            pl.BlockSpec((1, H), lambda i: (0, 0)),
        ],
        out_specs=pl.BlockSpec((blk, H), lambda i: (i, 0)),
        out_shape=jax.ShapeDtypeStruct((NPAD, H), jnp.float32),
    )(x, out, W, b.reshape(1, H))


def kernel(z, pos, edge_index, emb, W0, W1, W2, b0, b1, b2):
    z2 = jnp.pad(z.astype(jnp.int32), (0, NPAD - N)).reshape(ZW, 128)
    row = edge_index[0].astype(jnp.int32)
    col = edge_index[1].astype(jnp.int32)
    row2 = jnp.pad(row, (0, EPAD - E)).reshape(EW, 128)
    col2 = jnp.pad(col, (0, EPAD - E)).reshape(EW, 128)
    px = jnp.asarray(pos[:, 0])
    py = jnp.asarray(pos[:, 1])
    pz = jnp.asarray(pos[:, 2])
    zeros = jnp.zeros((ACC_ROWS // 16, H), jnp.float32)

    x = _embed(z2, emb)
    rbf2 = _rbf(row2, col2, px, py, pz)
    for W, b in ((W0, b0), (W1, b1), (W2, b2)):
        out = _mp(x, row2, col2, rbf2, zeros)
        x = _mm(x, out, W, b)
    return x[:N]


# speedup vs baseline: 1.0005x; 1.0005x over previous
import dataclasses
import functools

import jax
import jax.numpy as jnp
from jax import lax
from jax.experimental import pallas as pl
from jax.experimental.pallas import tpu as pltpu
from jax.experimental.pallas import tpu_sc as plsc

N = 50000
E = 800000
H = 64

HALF = 25088
NPAD = 2 * HALF
ACC_ROWS = 25216
ZW = NPAD // 128
ZCH = 49

EPAD = 819200
EW = EPAD // 128
WPT_B = EW // 16
CH_B = 8
NCH_B = WPT_B // CH_B
SW = 2 * CH_B
WPT_A = EW // 32
CH_A = 8
NCH_A = WPT_A // CH_A

_MESH = plsc.VectorSubcoreMesh(
    core_axis_name="c", subcore_axis_name="s", num_cores=2, num_subcores=16)

_CP = pltpu.CompilerParams(
    needs_layout_passes=False, use_tc_tiling_on_sc=False)


def _embed_body(z_hbm, emb_hbm, x_hbm, zv, gb):
    wid = lax.axis_index("s") * 2 + lax.axis_index("c")
    for i in range(2):
        ci = wid + 32 * i

        def _one(ci=ci):
            pltpu.sync_copy(z_hbm.at[pl.ds(ci * 8, 8)], zv)
            for w in range(8):
                pltpu.sync_copy(emb_hbm.at[zv.at[w]], gb)
                pltpu.sync_copy(gb, x_hbm.at[pl.ds((ci * 8 + w) * 128, 128)])

        if i == 0:
            _one()
        else:
            pl.when(ci < ZCH)(_one)


def _embed(z2, emb):
    f = pl.kernel(
        _embed_body,
        out_type=jax.ShapeDtypeStruct((NPAD, H), jnp.float32),
        mesh=_MESH,
        compiler_params=_CP,
        scratch_types=[
            pltpu.VMEM((8, 128), jnp.int32),
            pltpu.VMEM((128, H), jnp.float32),
        ],
    )
    return f(z2, emb)


def _rsqrt(s):
    i = plsc.bitcast(s, jnp.int32)
    y = plsc.bitcast(jnp.int32(0x5F3759DF) - (i >> 1), jnp.float32)
    for _ in range(3):
        y = y * (1.5 - 0.5 * s * y * y)
    return y


def _rbf_body(row_hbm, col_hbm, px_hbm, py_hbm, pz_hbm, rbf_hbm,
              rv, cv, bxr, byr, bzr, bxc, byc, bzc, ob, sem):
    wid = lax.axis_index("s") * 2 + lax.axis_index("c")
    base_w = wid * WPT_A

    @pl.loop(0, NCH_A)
    def _chunk(ch):
        w0 = base_w + ch * CH_A
        pltpu.sync_copy(row_hbm.at[pl.ds(w0, CH_A)], rv)
        pltpu.sync_copy(col_hbm.at[pl.ds(w0, CH_A)], cv)
        cps = []
        for w in range(CH_A):
            cps.append(pltpu.async_copy(px_hbm.at[rv.at[w]], bxr.at[w], sem))
            cps.append(pltpu.async_copy(py_hbm.at[rv.at[w]], byr.at[w], sem))
            cps.append(pltpu.async_copy(pz_hbm.at[rv.at[w]], bzr.at[w], sem))
            cps.append(pltpu.async_copy(px_hbm.at[cv.at[w]], bxc.at[w], sem))
            cps.append(pltpu.async_copy(py_hbm.at[cv.at[w]], byc.at[w], sem))
            cps.append(pltpu.async_copy(pz_hbm.at[cv.at[w]], bzc.at[w], sem))
        for cp in cps:
            cp.wait()
        for w in range(CH_A):
            for j in range(0, 128, 16):
                sl = (w, pl.ds(j, 16))
                dx = bxr[sl] - bxc[sl]
                dy = byr[sl] - byc[sl]
                dz = bzr[sl] - bzc[sl]
                s = dx * dx + dy * dy + dz * dz + 1e-12
                r = jnp.exp(-(s * _rsqrt(s)))
                eidx = (w0 + w) * 128 + j + lax.iota(jnp.int32, 16)
                ob[sl] = jnp.where(eidx < E, r, 0.0)
        pltpu.sync_copy(ob, rbf_hbm.at[pl.ds(w0, CH_A)])


def _rbf(row2, col2, px, py, pz):
    f = pl.kernel(
        _rbf_body,
        out_type=jax.ShapeDtypeStruct((EW, 128), jnp.float32),
        mesh=_MESH,
        compiler_params=_CP,
        scratch_types=(
            [pltpu.VMEM((CH_A, 128), jnp.int32)] * 2
            + [pltpu.VMEM((CH_A, 128), jnp.float32)] * 7
            + [pltpu.SemaphoreType.DMA]
        ),
    )
    return f(row2, col2, px, py, pz)


def _mp_body(x_hbm, row_hbm, col_hbm, rbf_hbm, zero_hbm, out_hbm,
             rv, cv, fv, iv2, gb, acc,
             g0, g1, g2, g3, t0, t1, t2, t3):
    gsem = (g0, g1, g2, g3)
    tsem = (t0, t1, t2, t3)
    c = lax.axis_index("c")
    s = lax.axis_index("s")
    nbase = c * HALF
    zrows = ACC_ROWS // 16
    pltpu.sync_copy(zero_hbm.at[pl.ds(0, zrows)],
                    acc.at[pl.ds(s * zrows, zrows)])
    plsc.subcore_barrier()
    w_base = s * WPT_B

    @pl.loop(0, NCH_B)
    def _chunk(ch):
        w0 = w_base + ch * CH_B
        pltpu.sync_copy(row_hbm.at[pl.ds(w0, CH_B)], rv)
        pltpu.sync_copy(col_hbm.at[pl.ds(w0, CH_B)], cv)
        gps = [None] * 4
        for q in range(3):
            gps[q] = pltpu.async_copy(
                x_hbm.at[cv.at[q // 2, pl.ds((q % 2) * 64, 64)]],
                gb.at[q], gsem[q])
        pltpu.sync_copy(rbf_hbm.at[pl.ds(w0, CH_B)], fv)
        for q in range(SW):
            w8, h = q // 2, (q % 2) * 64
            for j in range(0, 64, 16):
                r16 = rv[w8, pl.ds(h + j, 16)]
                loc = r16 - nbase
                ok = (loc >= 0) & (loc < HALF)
                trash = HALF + h + j + lax.iota(jnp.int32, 16)
                iv2[q, pl.ds(j, 16)] = jnp.where(ok, loc, trash)
        scs = [None] * 4
        for q in range(SW):
            b = q % 4
            nq = q + 3
            if nq < SW:
                nb = nq % 4
                if scs[nb] is not None:
                    scs[nb].wait()
                gps[nb] = pltpu.async_copy(
                    x_hbm.at[cv.at[nq // 2, pl.ds((nq % 2) * 64, 64)]],
                    gb.at[nb], gsem[nb])
            gps[b].wait()

            @pl.loop(0, 64, step=16)
            def _scale(j, b=b, q=q):
                f16 = fv[q // 2, pl.ds((q % 2) * 64 + j, 16)]
                for u in range(16):
                    rb = f16[u]
                    for k in range(4):
                        sl = (b, j + u, pl.ds(k * 16, 16))
                        gb[sl] = gb[sl] * rb

            scs[b] = pltpu.async_copy(gb.at[b], acc.at[iv2.at[q]],
                                      tsem[b], add=True)
        for b in range(4):
            scs[b].wait()

    plsc.subcore_barrier()
    rpt = HALF // 16
    pltpu.sync_copy(acc.at[pl.ds(s * rpt, rpt)],
                    out_hbm.at[pl.ds(nbase + s * rpt, rpt)])


def _mp(x, row2, col2, rbf2, zeros):
    f = pl.kernel(
        _mp_body,
        out_type=jax.ShapeDtypeStruct((NPAD, H), jnp.float32),
        mesh=_MESH,
        compiler_params=_CP,
        scratch_types=[
            pltpu.VMEM((CH_B, 128), jnp.int32),
            pltpu.VMEM((CH_B, 128), jnp.int32),
            pltpu.VMEM((CH_B, 128), jnp.float32),
            pltpu.VMEM((SW, 64), jnp.int32),
            pltpu.VMEM((4, 64, H), jnp.float32),
            pltpu.VMEM_SHARED((ACC_ROWS, H), jnp.float32),
        ] + [pltpu.SemaphoreType.DMA] * 8,
    )
    return f(x, row2, col2, rbf2, zeros)


def _mm_body(x_ref, o_ref, w_ref, b_ref, out_ref):
    acc = x_ref[...] + o_ref[...]
    out_ref[...] = jnp.maximum(
        lax.dot_general(acc, w_ref[...], (((1,), (1,)), ((), ())),
                        preferred_element_type=jnp.float32) + b_ref[...], 0.0)


def _mm(x, out, W, b):
    blk = 1024
    return pl.pallas_call(
        _mm_body,
        grid=(NPAD // blk,),
        in_specs=[
            pl.BlockSpec((blk, H), lambda i: (i, 0)),
            pl.BlockSpec((blk, H), lambda i: (i, 0)),
            pl.BlockSpec((H, H), lambda i: (0, 0)),
            pl.BlockSpec((1, H), lambda i: (0, 0)),
        ],
        out_specs=pl.BlockSpec((blk, H), lambda i: (i, 0)),
        out_shape=jax.ShapeDtypeStruct((NPAD, H), jnp.float32),
    )(x, out, W, b.reshape(1, H))


def kernel(z, pos, edge_index, emb, W0, W1, W2, b0, b1, b2):
    z2 = jnp.pad(z.astype(jnp.int32), (0, NPAD - N)).reshape(ZW, 128)
    row = edge_index[0].astype(jnp.int32)
    col = edge_index[1].astype(jnp.int32)
    row2 = jnp.pad(row, (0, EPAD - E)).reshape(EW, 128)
    col2 = jnp.pad(col, (0, EPAD - E)).reshape(EW, 128)
    px = jnp.asarray(pos[:, 0])
    py = jnp.asarray(pos[:, 1])
    pz = jnp.asarray(pos[:, 2])
    zeros = jnp.zeros((ACC_ROWS // 16, H), jnp.float32)

    x = _embed(z2, emb)
    rbf2 = _rbf(row2, col2, px, py, pz)
    for W, b in ((W0, b0), (W1, b1), (W2, b2)):
        out = _mp(x, row2, col2, rbf2, zeros)
        x = _mm(x, out, W, b)
    return x[:N]


# speedup vs baseline: 1.0190x; 1.0185x over previous
import dataclasses
import functools

import jax
import jax.numpy as jnp
from jax import lax
from jax.experimental import pallas as pl
from jax.experimental.pallas import tpu as pltpu
from jax.experimental.pallas import tpu_sc as plsc

N = 50000
E = 800000
H = 64

HALF = 25088
NPAD = 2 * HALF
ACC_ROWS = 25216
ZW = NPAD // 128
ZCH = 49

EPAD = 819200
EW = EPAD // 128
WPT_B = EW // 16
CH_B = 16
NCH_B = WPT_B // CH_B
WPT_A = EW // 32
CH_A = 8
NCH_A = WPT_A // CH_A

_MESH = plsc.VectorSubcoreMesh(
    core_axis_name="c", subcore_axis_name="s", num_cores=2, num_subcores=16)

_CP = pltpu.CompilerParams(
    needs_layout_passes=False, use_tc_tiling_on_sc=False)


def _embed_body(z_hbm, emb_hbm, x_hbm, zv, gb):
    wid = lax.axis_index("s") * 2 + lax.axis_index("c")
    for i in range(2):
        ci = wid + 32 * i

        def _one(ci=ci):
            pltpu.sync_copy(z_hbm.at[pl.ds(ci * 8, 8)], zv)
            for w in range(8):
                pltpu.sync_copy(emb_hbm.at[zv.at[w]], gb)
                pltpu.sync_copy(gb, x_hbm.at[pl.ds((ci * 8 + w) * 128, 128)])

        if i == 0:
            _one()
        else:
            pl.when(ci < ZCH)(_one)


def _embed(z2, emb):
    f = pl.kernel(
        _embed_body,
        out_type=jax.ShapeDtypeStruct((NPAD, H), jnp.float32),
        mesh=_MESH,
        compiler_params=_CP,
        scratch_types=[
            pltpu.VMEM((8, 128), jnp.int32),
            pltpu.VMEM((128, H), jnp.float32),
        ],
    )
    return f(z2, emb)


def _rsqrt(s):
    i = plsc.bitcast(s, jnp.int32)
    y = plsc.bitcast(jnp.int32(0x5F3759DF) - (i >> 1), jnp.float32)
    for _ in range(3):
        y = y * (1.5 - 0.5 * s * y * y)
    return y


def _rbf_body(row_hbm, col_hbm, px_hbm, py_hbm, pz_hbm, rbf_hbm,
              rv, cv, bxr, byr, bzr, bxc, byc, bzc, ob, sem):
    wid = lax.axis_index("s") * 2 + lax.axis_index("c")
    base_w = wid * WPT_A

    @pl.loop(0, NCH_A)
    def _chunk(ch):
        w0 = base_w + ch * CH_A
        pltpu.sync_copy(row_hbm.at[pl.ds(w0, CH_A)], rv)
        pltpu.sync_copy(col_hbm.at[pl.ds(w0, CH_A)], cv)
        cps = []
        for w in range(CH_A):
            cps.append(pltpu.async_copy(px_hbm.at[rv.at[w]], bxr.at[w], sem))
            cps.append(pltpu.async_copy(py_hbm.at[rv.at[w]], byr.at[w], sem))
            cps.append(pltpu.async_copy(pz_hbm.at[rv.at[w]], bzr.at[w], sem))
            cps.append(pltpu.async_copy(px_hbm.at[cv.at[w]], bxc.at[w], sem))
            cps.append(pltpu.async_copy(py_hbm.at[cv.at[w]], byc.at[w], sem))
            cps.append(pltpu.async_copy(pz_hbm.at[cv.at[w]], bzc.at[w], sem))
        for cp in cps:
            cp.wait()
        for w in range(CH_A):
            for j in range(0, 128, 16):
                sl = (w, pl.ds(j, 16))
                dx = bxr[sl] - bxc[sl]
                dy = byr[sl] - byc[sl]
                dz = bzr[sl] - bzc[sl]
                s = dx * dx + dy * dy + dz * dz + 1e-12
                r = jnp.exp(-(s * _rsqrt(s)))
                eidx = (w0 + w) * 128 + j + lax.iota(jnp.int32, 16)
                ob[sl] = jnp.where(eidx < E, r, 0.0)
        pltpu.sync_copy(ob, rbf_hbm.at[pl.ds(w0, CH_A)])


def _rbf(row2, col2, px, py, pz):
    f = pl.kernel(
        _rbf_body,
        out_type=jax.ShapeDtypeStruct((EW, 128), jnp.float32),
        mesh=_MESH,
        compiler_params=_CP,
        scratch_types=(
            [pltpu.VMEM((CH_A, 128), jnp.int32)] * 2
            + [pltpu.VMEM((CH_A, 128), jnp.float32)] * 7
            + [pltpu.SemaphoreType.DMA]
        ),
    )
    return f(row2, col2, px, py, pz)


def _mp_body(x_hbm, row_hbm, col_hbm, rbf_hbm, zero_hbm, out_hbm,
             rv, cv, fv, iv, gb, acc, s0, s1, t0, t1):
    sems = (s0, s1)
    ssems = (t0, t1)
    c = lax.axis_index("c")
    s = lax.axis_index("s")
    nbase = c * HALF
    zrows = ACC_ROWS // 16
    pltpu.sync_copy(zero_hbm.at[pl.ds(0, zrows)],
                    acc.at[pl.ds(s * zrows, zrows)])
    plsc.subcore_barrier()
    w_base = s * WPT_B

    @pl.loop(0, NCH_B)
    def _chunk(ch):
        w0 = w_base + ch * CH_B
        pltpu.sync_copy(row_hbm.at[pl.ds(w0, CH_B)], rv)
        pltpu.sync_copy(col_hbm.at[pl.ds(w0, CH_B)], cv)
        cps = [None, None]
        cps[0] = pltpu.async_copy(x_hbm.at[cv.at[0]], gb.at[0], sems[0])
        pltpu.sync_copy(rbf_hbm.at[pl.ds(w0, CH_B)], fv)
        for j in range(0, 128, 16):
            for w in range(CH_B):
                r16 = rv[w, pl.ds(j, 16)]
                loc = r16 - nbase
                ok = (loc >= 0) & (loc < HALF)
                iv[w, pl.ds(j, 16)] = jnp.where(ok, loc, HALF)
        scs = [None, None]
        for w in range(CH_B):
            b = w % 2
            if w + 1 < CH_B:
                nb = 1 - b
                if w >= 1:
                    scs[nb].wait()
                cps[nb] = pltpu.async_copy(
                    x_hbm.at[cv.at[w + 1]], gb.at[nb], sems[nb])
            cps[b].wait()

            @pl.loop(0, 128, step=16)
            def _scale(j, b=b, w=w):
                f16 = fv[w, pl.ds(j, 16)]
                for u in range(16):
                    rb = f16[u]
                    for k in range(4):
                        sl = (b, j + u, pl.ds(k * 16, 16))
                        gb[sl] = gb[sl] * rb

            scs[b] = pltpu.async_copy(gb.at[b], acc.at[iv.at[w]],
                                      ssems[b], add=True)
        scs[0].wait()
        scs[1].wait()

    plsc.subcore_barrier()
    rpt = HALF // 16
    pltpu.sync_copy(acc.at[pl.ds(s * rpt, rpt)],
                    out_hbm.at[pl.ds(nbase + s * rpt, rpt)])


def _mp(x, row2, col2, rbf2, zeros):
    f = pl.kernel(
        _mp_body,
        out_type=jax.ShapeDtypeStruct((NPAD, H), jnp.float32),
        mesh=_MESH,
        compiler_params=_CP,
        scratch_types=[
            pltpu.VMEM((CH_B, 128), jnp.int32),
            pltpu.VMEM((CH_B, 128), jnp.int32),
            pltpu.VMEM((CH_B, 128), jnp.float32),
            pltpu.VMEM((CH_B, 128), jnp.int32),
            pltpu.VMEM((2, 128, H), jnp.float32),
            pltpu.VMEM_SHARED((ACC_ROWS, H), jnp.float32),
        ] + [pltpu.SemaphoreType.DMA] * 4,
    )
    return f(x, row2, col2, rbf2, zeros)


def _mm_body(x_ref, o_ref, w_ref, b_ref, out_ref):
    acc = x_ref[...] + o_ref[...]
    out_ref[...] = jnp.maximum(
        lax.dot_general(acc, w_ref[...], (((1,), (1,)), ((), ())),
                        preferred_element_type=jnp.float32) + b_ref[...], 0.0)


def _mm(x, out, W, b):
    blk = 1024
    return pl.pallas_call(
        _mm_body,
        grid=(NPAD // blk,),
        in_specs=[
            pl.BlockSpec((blk, H), lambda i: (i, 0)),
            pl.BlockSpec((blk, H), lambda i: (i, 0)),
            pl.BlockSpec((H, H), lambda i: (0, 0)),
            pl.BlockSpec((1, H), lambda i: (0, 0)),
        ],
        out_specs=pl.BlockSpec((blk, H), lambda i: (i, 0)),
        out_shape=jax.ShapeDtypeStruct((NPAD, H), jnp.float32),
    )(x, out, W, b.reshape(1, H))


def kernel(z, pos, edge_index, emb, W0, W1, W2, b0, b1, b2):
    z2 = jnp.pad(z.astype(jnp.int32), (0, NPAD - N)).reshape(ZW, 128)
    row = edge_index[0].astype(jnp.int32)
    col = edge_index[1].astype(jnp.int32)
    row2 = jnp.pad(row, (0, EPAD - E)).reshape(EW, 128)
    col2 = jnp.pad(col, (0, EPAD - E)).reshape(EW, 128)
    px = jnp.asarray(pos[:, 0])
    py = jnp.asarray(pos[:, 1])
    pz = jnp.asarray(pos[:, 2])
    zeros = jnp.zeros((ACC_ROWS // 16, H), jnp.float32)

    x = _embed(z2, emb)
    rbf2 = _rbf(row2, col2, px, py, pz)
    for W, b in ((W0, b0), (W1, b1), (W2, b2)):
        out = _mp(x, row2, col2, rbf2, zeros)
        x = _mm(x, out, W, b)
    return x[:N]
